# baseline (speedup 1.0000x reference)
import functools

import jax
import jax.numpy as jnp
from jax import lax
from jax.experimental import pallas as pl
from jax.experimental.pallas import tpu as pltpu
from jax.experimental.pallas import tpu_sc as plsc

D = 256
SEQ = 20
N = 4096
EPS = 1e-6
NC, NS = 2, 16
NW = NC * NS
CPW = N // NW
CHUNK = 4
SPC = CHUNK * SEQ
NBUF = 4
NCHUNK = CPW // CHUNK
NCOL = D // 16
A1PAD = 32


def _sc_body(a1i_hbm, a2i_hbm, emb1_hbm, emb2_hbm, cos_hbm,
             idx_v, rows0_v, rows1_v, rows2_v, rows3_v, a1i_v, a1rows_v,
             cosv, sem0, sem1, sem2, sem3, sema):
  w = lax.axis_index("s") * NC + lax.axis_index("c")

  lane_iota = lax.iota(jnp.int32, 16)
  zeros = tuple(jnp.zeros((16,), jnp.float32) for _ in range(NCOL))
  bufs = ((rows0_v, sem0), (rows1_v, sem1), (rows2_v, sem2), (rows3_v, sem3))

  def allsum(v):
    for d in (8, 4, 2, 1):
      v = v + v.at[lane_iota ^ d].get(mode="promise_in_bounds")
    return v

  pltpu.sync_copy(a1i_hbm, a1i_v)
  pltpu.async_copy(emb1_hbm.at[a1i_v], a1rows_v, sema).wait()

  def a1_body(t, accs):
    return tuple(accs[k] + a1rows_v[t, pl.ds(16 * k, 16)] for k in range(NCOL))

  s1 = lax.fori_loop(0, SEQ, a1_body, zeros)
  d1v = s1[0] * s1[0]
  for k in range(1, NCOL):
    d1v = d1v + s1[k] * s1[k]
  den1 = allsum(d1v)

  pltpu.sync_copy(a2i_hbm.at[pl.ds(w * CPW * SEQ, CPW * SEQ)], idx_v)

  def start_gather(ci, rows, sem):
    pltpu.async_copy(emb2_hbm.at[idx_v.at[pl.ds(ci * SPC, SPC)]], rows, sem)

  def wait_gather(ci, rows, sem):
    pltpu.make_async_copy(
        emb2_hbm.at[idx_v.at[pl.ds(ci * SPC, SPC)]], rows, sem).wait()

  for b, (rows, sem) in enumerate(bufs):
    start_gather(b, rows, sem)

  def outer(i, carry):
    ci0 = NBUF * i
    num_acc = jnp.zeros((16,), jnp.float32)
    den_acc = jnp.zeros((16,), jnp.float32)
    for b, (rows, sem) in enumerate(bufs):
      ci = ci0 + b
      wait_gather(ci, rows, sem)
      for j in range(CHUNK):
        base = j * SEQ

        def seq_body(t, accs):
          return tuple(
              accs[k] + rows[base + t, pl.ds(16 * k, 16)] for k in range(NCOL))

        s2 = lax.fori_loop(0, SEQ, seq_body, zeros)
        nv = s2[0] * s1[0]
        dv = s2[0] * s2[0]
        for k in range(1, NCOL):
          nv = nv + s2[k] * s1[k]
          dv = dv + s2[k] * s2[k]
        lane = b * CHUNK + j
        num_acc = jnp.where(lane_iota == lane, allsum(nv), num_acc)
        den_acc = jnp.where(lane_iota == lane, allsum(dv) * den1, den_acc)

      @pl.when(ci + NBUF < NCHUNK)
      def _():
        start_gather(ci + NBUF, rows, sem)

    x = den_acc
    y = lax.bitcast_convert_type(
        jnp.int32(0x5F3759DF) - (lax.bitcast_convert_type(x, jnp.int32) >> 1),
        jnp.float32)
    for _nw in range(3):
      y = y * (1.5 - 0.5 * x * y * y)
    cosv[pl.ds(16 * i, 16)] = num_acc / jnp.maximum(x * y, 400.0 * EPS)
    return None

  lax.fori_loop(0, NCHUNK // NBUF, outer, None)

  pltpu.sync_copy(cosv, cos_hbm.at[pl.ds(w * CPW, CPW)])


_sc_kernel = functools.partial(
    pl.kernel,
    mesh=plsc.VectorSubcoreMesh(core_axis_name="c", subcore_axis_name="s"),
    out_type=jax.ShapeDtypeStruct((N,), jnp.float32),
    scratch_types=[
        pltpu.VMEM((CPW * SEQ,), jnp.int32),
        pltpu.VMEM((SPC, D), jnp.float32),
        pltpu.VMEM((SPC, D), jnp.float32),
        pltpu.VMEM((SPC, D), jnp.float32),
        pltpu.VMEM((SPC, D), jnp.float32),
        pltpu.VMEM((A1PAD,), jnp.int32),
        pltpu.VMEM((A1PAD, D), jnp.float32),
        pltpu.VMEM((CPW,), jnp.float32),
        pltpu.SemaphoreType.DMA,
        pltpu.SemaphoreType.DMA,
        pltpu.SemaphoreType.DMA,
        pltpu.SemaphoreType.DMA,
        pltpu.SemaphoreType.DMA,
    ],
)(_sc_body)


def kernel(action1, actions2, emb1, emb2):
  a1p = jnp.concatenate([action1, jnp.zeros((A1PAD - SEQ,), jnp.int32)])
  a2r = actions2.reshape(-1)
  return _sc_kernel(a1p, a2r, emb1, emb2)


# speedup vs baseline: 1.1037x; 1.1037x over previous
import functools

import jax
import jax.numpy as jnp
from jax import lax
from jax.experimental import pallas as pl
from jax.experimental.pallas import tpu as pltpu
from jax.experimental.pallas import tpu_sc as plsc

D = 256
SEQ = 20
N = 4096
EPS = 1e-6
NC, NS = 2, 16
NW = NC * NS
CPW = N // NW
CHUNK = 4
SPC = CHUNK * SEQ
NBUF = 4
NCHUNK = CPW // CHUNK
NCOL = D // 16
A1PAD = 32
VOCAB = 100000


def _sc_body(a1i_hbm, a2i_hbm, emb1_hbm, emb2_hbm, cos_hbm,
             idx_v, rows0_v, rows1_v, rows2_v, rows3_v, a1i_v, a1rows_v,
             cosv, sem0, sem1, sem2, sem3, sema):
  w = lax.axis_index("s") * NC + lax.axis_index("c")

  lane_iota = lax.iota(jnp.int32, 16)
  zeros = tuple(jnp.zeros((16,), jnp.float32) for _ in range(NCOL))
  bufs = ((rows0_v, sem0), (rows1_v, sem1), (rows2_v, sem2), (rows3_v, sem3))

  def allsum(v):
    for d in (8, 4, 2, 1):
      v = v + v.at[lane_iota ^ d].get(mode="promise_in_bounds")
    return v

  pltpu.sync_copy(a1i_hbm, a1i_v.at[pl.ds(0, SEQ)])
  tail = a1i_v[pl.ds(16, 16)]
  a1i_v[pl.ds(16, 16)] = jnp.minimum(jnp.maximum(tail, 0), VOCAB - 1)
  pltpu.async_copy(emb1_hbm.at[a1i_v], a1rows_v, sema).wait()

  def a1_body(t, accs):
    return tuple(accs[k] + a1rows_v[t, pl.ds(16 * k, 16)] for k in range(NCOL))

  s1 = lax.fori_loop(0, SEQ, a1_body, zeros)
  d1v = s1[0] * s1[0]
  for k in range(1, NCOL):
    d1v = d1v + s1[k] * s1[k]
  den1 = allsum(d1v)

  pltpu.sync_copy(a2i_hbm.at[w], idx_v)

  def start_gather(ci, rows, sem):
    pltpu.async_copy(emb2_hbm.at[idx_v.at[ci]], rows, sem)

  def wait_gather(ci, rows, sem):
    pltpu.make_async_copy(emb2_hbm.at[idx_v.at[ci]], rows, sem).wait()

  for b, (rows, sem) in enumerate(bufs):
    start_gather(b, rows, sem)

  def outer(i, carry):
    ci0 = NBUF * i
    num_acc = jnp.zeros((16,), jnp.float32)
    den_acc = jnp.zeros((16,), jnp.float32)
    for b, (rows, sem) in enumerate(bufs):
      ci = ci0 + b
      wait_gather(ci, rows, sem)
      for j in range(CHUNK):
        base = j * SEQ

        def seq_body(t, accs):
          return tuple(
              accs[k] + rows[base + t, pl.ds(16 * k, 16)] for k in range(NCOL))

        s2 = lax.fori_loop(0, SEQ, seq_body, zeros)
        nv = s2[0] * s1[0]
        dv = s2[0] * s2[0]
        for k in range(1, NCOL):
          nv = nv + s2[k] * s1[k]
          dv = dv + s2[k] * s2[k]
        lane = b * CHUNK + j
        num_acc = jnp.where(lane_iota == lane, allsum(nv), num_acc)
        den_acc = jnp.where(lane_iota == lane, allsum(dv) * den1, den_acc)

      @pl.when(ci + NBUF < NCHUNK)
      def _():
        start_gather(ci + NBUF, rows, sem)

    x = den_acc
    y = lax.bitcast_convert_type(
        jnp.int32(0x5F3759DF) - (lax.bitcast_convert_type(x, jnp.int32) >> 1),
        jnp.float32)
    for _nw in range(3):
      y = y * (1.5 - 0.5 * x * y * y)
    cosv[pl.ds(16 * i, 16)] = num_acc / jnp.maximum(x * y, 400.0 * EPS)
    return None

  lax.fori_loop(0, NCHUNK // NBUF, outer, None)

  pltpu.sync_copy(cosv, cos_hbm.at[pl.ds(w * CPW, CPW)])


_sc_kernel = functools.partial(
    pl.kernel,
    mesh=plsc.VectorSubcoreMesh(core_axis_name="c", subcore_axis_name="s"),
    out_type=jax.ShapeDtypeStruct((N,), jnp.float32),
    scratch_types=[
        pltpu.VMEM((NCHUNK, SPC), jnp.int32),
        pltpu.VMEM((SPC, D), jnp.float32),
        pltpu.VMEM((SPC, D), jnp.float32),
        pltpu.VMEM((SPC, D), jnp.float32),
        pltpu.VMEM((SPC, D), jnp.float32),
        pltpu.VMEM((A1PAD,), jnp.int32),
        pltpu.VMEM((A1PAD, D), jnp.float32),
        pltpu.VMEM((CPW,), jnp.float32),
        pltpu.SemaphoreType.DMA,
        pltpu.SemaphoreType.DMA,
        pltpu.SemaphoreType.DMA,
        pltpu.SemaphoreType.DMA,
        pltpu.SemaphoreType.DMA,
    ],
)(_sc_body)


def kernel(action1, actions2, emb1, emb2):
  a2r = actions2.reshape(NW, NCHUNK, SPC)
  return _sc_kernel(action1, a2r, emb1, emb2)


# speedup vs baseline: 1.1893x; 1.0775x over previous
import functools

import jax
import jax.numpy as jnp
from jax import lax
from jax.experimental import pallas as pl
from jax.experimental.pallas import tpu as pltpu
from jax.experimental.pallas import tpu_sc as plsc

D = 256
SEQ = 20
N = 4096
EPS = 1e-6
NC, NS = 2, 16
NW = NC * NS
CPW = N // NW
CHUNK = 4
SPC = CHUNK * SEQ
NBUF = 4
NCHUNK = CPW // CHUNK
NCOL = D // 16
A1PAD = 32
VOCAB = 100000


def _sc_body(a1i_hbm, a2i_hbm, emb1_hbm, emb2_hbm, cos_hbm,
             idx_v, rows0_v, rows1_v, rows2_v, rows3_v, a1i_v, a1rows_v,
             cosv, sem0, sem1, sem2, sem3, sema, semi):
  w = lax.axis_index("s") * NC + lax.axis_index("c")

  lane_iota = lax.iota(jnp.int32, 16)
  zeros = tuple(jnp.zeros((16,), jnp.float32) for _ in range(NCOL))
  bufs = ((rows0_v, sem0), (rows1_v, sem1), (rows2_v, sem2), (rows3_v, sem3))

  def allsum(v):
    for d in (8, 4, 2, 1):
      v = v + v.at[lane_iota ^ d].get(mode="promise_in_bounds")
    return v

  idx_cp = pltpu.make_async_copy(a2i_hbm.at[w], idx_v, semi)
  idx_cp.start()

  pltpu.sync_copy(a1i_hbm, a1i_v.at[pl.ds(0, SEQ)])
  tail = a1i_v[pl.ds(16, 16)]
  a1i_v[pl.ds(16, 16)] = jnp.minimum(jnp.maximum(tail, 0), VOCAB - 1)
  a1_cp = pltpu.make_async_copy(emb1_hbm.at[a1i_v], a1rows_v, sema)
  a1_cp.start()

  def start_gather(ci, rows, sem):
    pltpu.async_copy(emb2_hbm.at[idx_v.at[ci]], rows, sem)

  def wait_gather(ci, rows, sem):
    pltpu.make_async_copy(emb2_hbm.at[idx_v.at[ci]], rows, sem).wait()

  idx_cp.wait()
  for b, (rows, sem) in enumerate(bufs):
    start_gather(b, rows, sem)

  a1_cp.wait()

  def a1_body(t, accs):
    return tuple(accs[k] + a1rows_v[t, pl.ds(16 * k, 16)] for k in range(NCOL))

  s1 = lax.fori_loop(0, SEQ, a1_body, zeros)
  d1v = s1[0] * s1[0]
  for k in range(1, NCOL):
    d1v = d1v + s1[k] * s1[k]
  den1 = allsum(d1v)

  def outer(i, carry):
    ci0 = NBUF * i
    num_acc = jnp.zeros((16,), jnp.float32)
    den_acc = jnp.zeros((16,), jnp.float32)
    for b, (rows, sem) in enumerate(bufs):
      ci = ci0 + b
      wait_gather(ci, rows, sem)
      for j in range(CHUNK):
        base = j * SEQ

        def seq_body(t, accs):
          return tuple(
              accs[k] + rows[base + t, pl.ds(16 * k, 16)] for k in range(NCOL))

        s2 = lax.fori_loop(0, SEQ, seq_body, zeros)
        nv = s2[0] * s1[0]
        dv = s2[0] * s2[0]
        for k in range(1, NCOL):
          nv = nv + s2[k] * s1[k]
          dv = dv + s2[k] * s2[k]
        lane = b * CHUNK + j
        num_acc = jnp.where(lane_iota == lane, allsum(nv), num_acc)
        den_acc = jnp.where(lane_iota == lane, allsum(dv) * den1, den_acc)

      @pl.when(ci + NBUF < NCHUNK)
      def _():
        start_gather(ci + NBUF, rows, sem)

    x = den_acc
    y = lax.bitcast_convert_type(
        jnp.int32(0x5F3759DF) - (lax.bitcast_convert_type(x, jnp.int32) >> 1),
        jnp.float32)
    for _nw in range(3):
      y = y * (1.5 - 0.5 * x * y * y)
    cosv[pl.ds(16 * i, 16)] = num_acc / jnp.maximum(x * y, 400.0 * EPS)
    return None

  lax.fori_loop(0, NCHUNK // NBUF, outer, None)

  pltpu.sync_copy(cosv, cos_hbm.at[pl.ds(w * CPW, CPW)])


_sc_kernel = functools.partial(
    pl.kernel,
    mesh=plsc.VectorSubcoreMesh(core_axis_name="c", subcore_axis_name="s"),
    out_type=jax.ShapeDtypeStruct((N,), jnp.float32),
    scratch_types=[
        pltpu.VMEM((NCHUNK, SPC), jnp.int32),
        pltpu.VMEM((SPC, D), jnp.float32),
        pltpu.VMEM((SPC, D), jnp.float32),
        pltpu.VMEM((SPC, D), jnp.float32),
        pltpu.VMEM((SPC, D), jnp.float32),
        pltpu.VMEM((A1PAD,), jnp.int32),
        pltpu.VMEM((A1PAD, D), jnp.float32),
        pltpu.VMEM((CPW,), jnp.float32),
        pltpu.SemaphoreType.DMA,
        pltpu.SemaphoreType.DMA,
        pltpu.SemaphoreType.DMA,
        pltpu.SemaphoreType.DMA,
        pltpu.SemaphoreType.DMA,
        pltpu.SemaphoreType.DMA,
    ],
)(_sc_body)


def kernel(action1, actions2, emb1, emb2):
  a2r = actions2.reshape(NW, NCHUNK, SPC)
  return _sc_kernel(action1, a2r, emb1, emb2)
